# baseline (speedup 1.0000x reference)
import functools

import jax
import jax.numpy as jnp
from jax import lax
from jax.experimental import pallas as pl
from jax.experimental.pallas import tpu as pltpu
from jax.experimental.pallas import tpu_sc as plsc

_N, _E, _D = 10000, 320000, 128
_NC, _NS, _L = 2, 16, 16
_K = 96
_CH0 = 69
_CH1 = 141
_TOTCH = _NS * (_CH0 + _CH1)
_EPAD = _K * _TOTCH
_NSLOT = 6
_RPT = 624
_RTAIL = _N - _NS * _RPT

_sc_mesh = plsc.VectorSubcoreMesh(core_axis_name="c", subcore_axis_name="s")


@functools.partial(
    pl.kernel,
    out_type=jax.ShapeDtypeStruct((_NC, _N, _D), jnp.float32),
    mesh=_sc_mesh,
    scratch_types=[
        pltpu.VMEM((_NSLOT, _K), jnp.int32),
        pltpu.VMEM((_NSLOT, _K), jnp.int32),
        pltpu.VMEM((_NSLOT, _K), jnp.float32),
        pltpu.VMEM((_K, _D), jnp.float32),
        pltpu.VMEM((_K, _D), jnp.float32),
        pltpu.VMEM((_K, _D), jnp.float32),
        pltpu.VMEM_SHARED((_N, _D), jnp.float32),
        pltpu.SemaphoreType.DMA,
        pltpu.SemaphoreType.DMA,
        pltpu.SemaphoreType.DMA,
        pltpu.SemaphoreType.DMA,
        pltpu.SemaphoreType.DMA,
        pltpu.SemaphoreType.DMA,
        pltpu.SemaphoreType.DMA((_NSLOT,)),
    ],
    compiler_params=pltpu.CompilerParams(needs_layout_passes=False),
)
def _sc_scatter(x_hbm, src_hbm, dst_hbm, ew_hbm, zero_hbm, out_hbm,
                src_v, dst_v, ew_v, r0, r1, r2, acc_sh,
                gsem0, gsem1, gsem2, ssem0, ssem1, ssem2, esem):
    c = lax.axis_index("c")
    s = lax.axis_index("s")
    rbufs = (r0, r1, r2)
    gsems = (gsem0, gsem1, gsem2)
    ssems = (ssem0, ssem1, ssem2)

    nch = jnp.where(c == 0, _CH0, _CH1)
    cbase = jnp.where(c == 0, s * _CH0, _NS * _CH0 + s * _CH1)

    def efetch(m, slot):
        cid = cbase + m
        pltpu.async_copy(src_hbm.at[cid], src_v.at[slot], esem.at[slot])
        pltpu.async_copy(dst_hbm.at[cid], dst_v.at[slot], esem.at[slot])
        pltpu.async_copy(ew_hbm.at[cid], ew_v.at[slot], esem.at[slot])

    def ewait(slot):
        pltpu.make_async_copy(src_hbm.at[0], src_v.at[slot],
                              esem.at[slot]).wait()
        pltpu.make_async_copy(src_hbm.at[0], dst_v.at[slot],
                              esem.at[slot]).wait()
        pltpu.make_async_copy(ew_hbm.at[0], ew_v.at[slot],
                              esem.at[slot]).wait()

    pltpu.sync_copy(zero_hbm.at[pl.ds(s * _RPT, _RPT)],
                    acc_sh.at[pl.ds(s * _RPT, _RPT)])

    @pl.when(s == 0)
    def _zero_tail():
        pltpu.sync_copy(zero_hbm.at[pl.ds(_NS * _RPT, _RTAIL)],
                        acc_sh.at[pl.ds(_NS * _RPT, _RTAIL)])

    plsc.subcore_barrier()

    for j in range(4):
        efetch(j, j)
    ewait(0)
    pltpu.async_copy(x_hbm.at[src_v.at[0]], r0, gsem0)
    ewait(1)
    pltpu.async_copy(x_hbm.at[src_v.at[1]], r1, gsem1)

    @pl.loop(0, nch, step=3)
    def _chunk(g):
        for b in range(3):
            m = g + b
            rb = rbufs[b]
            sm = lax.rem(g, _NSLOT) + b

            pltpu.make_async_copy(x_hbm.at[src_v.at[0]], rb,
                                  gsems[b]).wait()

            @pl.loop(0, _K, unroll=4)
            def _scale(k):
                w = plsc.load_gather(
                    ew_v, [jnp.full((_L,), sm, jnp.int32),
                           jnp.full((_L,), k, jnp.int32)])
                for col in range(_D // _L):
                    sl = pl.ds(col * _L, _L)
                    rb[k, sl] = rb[k, sl] * w

            pltpu.async_copy(rb, acc_sh.at[dst_v.at[sm]], ssems[b],
                             add=True)

            @pl.when(m >= 1)
            def _drain():
                pltpu.make_async_copy(rbufs[(b + 2) % 3],
                                      acc_sh.at[dst_v.at[0]],
                                      ssems[(b + 2) % 3]).wait()

            @pl.when(m + 2 < nch)
            def _prefetch_gather():
                s2 = lax.rem(m + 2, _NSLOT)
                ewait(s2)
                pltpu.async_copy(x_hbm.at[src_v.at[s2]],
                                 rbufs[(b + 2) % 3], gsems[(b + 2) % 3])

            @pl.when(m + 4 < nch)
            def _prefetch_idx():
                efetch(m + 4, lax.rem(m + 4, _NSLOT))

    pltpu.make_async_copy(r2, acc_sh.at[dst_v.at[0]], ssem2).wait()

    plsc.subcore_barrier()
    pltpu.sync_copy(acc_sh.at[pl.ds(s * _RPT, _RPT)],
                    out_hbm.at[c, pl.ds(s * _RPT, _RPT)])

    @pl.when(s == 0)
    def _write_tail():
        pltpu.sync_copy(acc_sh.at[pl.ds(_NS * _RPT, _RTAIL)],
                        out_hbm.at[c, pl.ds(_NS * _RPT, _RTAIL)])


_BN = 1000


def _tc_post_body(p_ref, wt_ref, b_ref, m_ref, o_ref):
    a = p_ref[0] + p_ref[1]
    h = jnp.dot(a, wt_ref[...], preferred_element_type=jnp.float32)
    h = (h + b_ref[...]) * (m_ref[...] > 0).astype(jnp.float32)
    o_ref[...] = jnp.maximum(h, 0.0)


def _tc_post(p, wt, b, m_col):
    return pl.pallas_call(
        _tc_post_body,
        grid=(_N // _BN,),
        in_specs=[
            pl.BlockSpec((_NC, _BN, _D), lambda i: (0, i, 0)),
            pl.BlockSpec((_D, _D), lambda i: (0, 0)),
            pl.BlockSpec((1, _D), lambda i: (0, 0)),
            pl.BlockSpec((_BN, 1), lambda i: (i, 0)),
        ],
        out_specs=pl.BlockSpec((_BN, _D), lambda i: (i, 0)),
        out_shape=jax.ShapeDtypeStruct((_N, _D), jnp.float32),
    )(p, wt, b, m_col)


@jax.jit
def _run(x, edge_index_list, edge_weight_list, mask, W1, b1, W2, b2):
    x0 = x[0]
    src = edge_index_list[0, 0]
    dst = edge_index_list[0, 1]
    ew = edge_weight_list[0]
    pad = _EPAD - _E
    shape2 = (_TOTCH, _K)
    src_p = jnp.pad(src, (0, pad)).reshape(shape2)
    dst_p = jnp.pad(dst, (0, pad)).reshape(shape2)
    ew_p = jnp.pad(ew, (0, pad)).reshape(shape2)
    zero = jnp.zeros((_N, _D), jnp.float32)
    m_col = mask[0][:, None]

    p1 = _sc_scatter(x0, src_p, dst_p, ew_p, zero)
    o1 = _tc_post(p1, W1.T, b1[None], m_col)
    p2 = _sc_scatter(o1, src_p, dst_p, ew_p, zero)
    o2 = _tc_post(p2, W2.T, b2[None], m_col)
    return o2[None]


def kernel(x, edge_index_list, edge_weight_list, mask, W1, b1, W2, b2):
    return _run(x, edge_index_list, edge_weight_list, mask, W1, b1, W2, b2)


# speedup vs baseline: 1.0345x; 1.0345x over previous
import functools

import jax
import jax.numpy as jnp
from jax import lax
from jax.experimental import pallas as pl
from jax.experimental.pallas import tpu as pltpu
from jax.experimental.pallas import tpu_sc as plsc

_N, _E, _D = 10000, 320000, 128
_NC, _NS, _L = 2, 16, 16
_K = 96
_CH0 = 105
_CH1 = 105
_TOTCH = _NS * (_CH0 + _CH1)
_EPAD = _K * _TOTCH
_NSLOT = 6
_RPT = 624
_RTAIL = _N - _NS * _RPT

_sc_mesh = plsc.VectorSubcoreMesh(core_axis_name="c", subcore_axis_name="s")


@functools.partial(
    pl.kernel,
    out_type=jax.ShapeDtypeStruct((_NC, _N, _D), jnp.float32),
    mesh=_sc_mesh,
    scratch_types=[
        pltpu.VMEM((_NSLOT, _K), jnp.int32),
        pltpu.VMEM((_NSLOT, _K), jnp.int32),
        pltpu.VMEM((_NSLOT, _K), jnp.float32),
        pltpu.VMEM((_K, _D), jnp.float32),
        pltpu.VMEM((_K, _D), jnp.float32),
        pltpu.VMEM((_K, _D), jnp.float32),
        pltpu.VMEM_SHARED((_N, _D), jnp.float32),
        pltpu.SemaphoreType.DMA,
        pltpu.SemaphoreType.DMA,
        pltpu.SemaphoreType.DMA,
        pltpu.SemaphoreType.DMA,
        pltpu.SemaphoreType.DMA,
        pltpu.SemaphoreType.DMA,
        pltpu.SemaphoreType.DMA((_NSLOT,)),
    ],
    compiler_params=pltpu.CompilerParams(needs_layout_passes=False),
)
def _sc_scatter(x_hbm, src_hbm, dst_hbm, ew_hbm, zero_hbm, out_hbm,
                src_v, dst_v, ew_v, r0, r1, r2, acc_sh,
                gsem0, gsem1, gsem2, ssem0, ssem1, ssem2, esem):
    c = lax.axis_index("c")
    s = lax.axis_index("s")
    rbufs = (r0, r1, r2)
    gsems = (gsem0, gsem1, gsem2)
    ssems = (ssem0, ssem1, ssem2)

    nch = jnp.where(c == 0, _CH0, _CH1)
    cbase = jnp.where(c == 0, s * _CH0, _NS * _CH0 + s * _CH1)

    def efetch(m, slot):
        cid = cbase + m
        pltpu.async_copy(src_hbm.at[cid], src_v.at[slot], esem.at[slot])
        pltpu.async_copy(dst_hbm.at[cid], dst_v.at[slot], esem.at[slot])
        pltpu.async_copy(ew_hbm.at[cid], ew_v.at[slot], esem.at[slot])

    def ewait(slot):
        pltpu.make_async_copy(src_hbm.at[0], src_v.at[slot],
                              esem.at[slot]).wait()
        pltpu.make_async_copy(src_hbm.at[0], dst_v.at[slot],
                              esem.at[slot]).wait()
        pltpu.make_async_copy(ew_hbm.at[0], ew_v.at[slot],
                              esem.at[slot]).wait()

    pltpu.sync_copy(zero_hbm.at[pl.ds(s * _RPT, _RPT)],
                    acc_sh.at[pl.ds(s * _RPT, _RPT)])

    @pl.when(s == 0)
    def _zero_tail():
        pltpu.sync_copy(zero_hbm.at[pl.ds(_NS * _RPT, _RTAIL)],
                        acc_sh.at[pl.ds(_NS * _RPT, _RTAIL)])

    plsc.subcore_barrier()

    for j in range(4):
        efetch(j, j)
    ewait(0)
    pltpu.async_copy(x_hbm.at[src_v.at[0]], r0, gsem0)
    ewait(1)
    pltpu.async_copy(x_hbm.at[src_v.at[1]], r1, gsem1)

    @pl.loop(0, nch, step=3)
    def _chunk(g):
        for b in range(3):
            m = g + b
            rb = rbufs[b]
            sm = lax.rem(g, _NSLOT) + b

            pltpu.make_async_copy(x_hbm.at[src_v.at[0]], rb,
                                  gsems[b]).wait()

            @pl.when(m >= 1)
            def _drain():
                pltpu.make_async_copy(rbufs[(b + 2) % 3],
                                      acc_sh.at[dst_v.at[0]],
                                      ssems[(b + 2) % 3]).wait()

            @pl.when(m + 2 < nch)
            def _prefetch_gather():
                s2 = lax.rem(m + 2, _NSLOT)
                ewait(s2)
                pltpu.async_copy(x_hbm.at[src_v.at[s2]],
                                 rbufs[(b + 2) % 3], gsems[(b + 2) % 3])

            @pl.when(m + 4 < nch)
            def _prefetch_idx():
                efetch(m + 4, lax.rem(m + 4, _NSLOT))

            @pl.loop(0, _K, unroll=4)
            def _scale(k):
                w = plsc.load_gather(
                    ew_v, [jnp.full((_L,), sm, jnp.int32),
                           jnp.full((_L,), k, jnp.int32)])
                for col in range(_D // _L):
                    sl = pl.ds(col * _L, _L)
                    rb[k, sl] = rb[k, sl] * w

            pltpu.async_copy(rb, acc_sh.at[dst_v.at[sm]], ssems[b],
                             add=True)

    pltpu.make_async_copy(r2, acc_sh.at[dst_v.at[0]], ssem2).wait()

    plsc.subcore_barrier()
    pltpu.sync_copy(acc_sh.at[pl.ds(s * _RPT, _RPT)],
                    out_hbm.at[c, pl.ds(s * _RPT, _RPT)])

    @pl.when(s == 0)
    def _write_tail():
        pltpu.sync_copy(acc_sh.at[pl.ds(_NS * _RPT, _RTAIL)],
                        out_hbm.at[c, pl.ds(_NS * _RPT, _RTAIL)])


_BN = 1000


def _tc_post_body(p_ref, wt_ref, b_ref, m_ref, o_ref):
    a = p_ref[0] + p_ref[1]
    h = jnp.dot(a, wt_ref[...], preferred_element_type=jnp.float32)
    h = (h + b_ref[...]) * (m_ref[...] > 0).astype(jnp.float32)
    o_ref[...] = jnp.maximum(h, 0.0)


def _tc_post(p, wt, b, m_col):
    return pl.pallas_call(
        _tc_post_body,
        grid=(_N // _BN,),
        in_specs=[
            pl.BlockSpec((_NC, _BN, _D), lambda i: (0, i, 0)),
            pl.BlockSpec((_D, _D), lambda i: (0, 0)),
            pl.BlockSpec((1, _D), lambda i: (0, 0)),
            pl.BlockSpec((_BN, 1), lambda i: (i, 0)),
        ],
        out_specs=pl.BlockSpec((_BN, _D), lambda i: (i, 0)),
        out_shape=jax.ShapeDtypeStruct((_N, _D), jnp.float32),
    )(p, wt, b, m_col)


@jax.jit
def _run(x, edge_index_list, edge_weight_list, mask, W1, b1, W2, b2):
    x0 = x[0]
    src = edge_index_list[0, 0]
    dst = edge_index_list[0, 1]
    ew = edge_weight_list[0]
    pad = _EPAD - _E
    shape2 = (_TOTCH, _K)
    src_p = jnp.pad(src, (0, pad)).reshape(shape2)
    dst_p = jnp.pad(dst, (0, pad)).reshape(shape2)
    ew_p = jnp.pad(ew, (0, pad)).reshape(shape2)
    zero = jnp.zeros((_N, _D), jnp.float32)
    m_col = mask[0][:, None]

    p1 = _sc_scatter(x0, src_p, dst_p, ew_p, zero)
    o1 = _tc_post(p1, W1.T, b1[None], m_col)
    p2 = _sc_scatter(o1, src_p, dst_p, ew_p, zero)
    o2 = _tc_post(p2, W2.T, b2[None], m_col)
    return o2[None]


def kernel(x, edge_index_list, edge_weight_list, mask, W1, b1, W2, b2):
    return _run(x, edge_index_list, edge_weight_list, mask, W1, b1, W2, b2)
